# baseline (speedup 1.0000x reference)
import functools

import jax
import jax.numpy as jnp
from jax import lax
from jax.experimental import pallas as pl
from jax.experimental.pallas import tpu as pltpu
from jax.experimental.pallas import tpu_sc as plsc

B, L, H, W, C = 16, 4096, 512, 512, 32
HID = H // 2

NC, NS, LANES = 2, 16, 16
NW = NC * NS
CW = 64

NSPLIT = 2
BSL = B // NSPLIT

CS_CH = 512
MLP_BLK = 512


def _cumsum_body(x_ref, out_ref, carry_ref):
    c = pl.program_id(1)

    @pl.when(c == 0)
    def _():
        carry_ref[...] = jnp.zeros_like(carry_ref)

    x = x_ref[0]
    inc = x
    k = 1
    while k < CS_CH:
        inc = inc + jnp.concatenate(
            [jnp.zeros((k, H), jnp.float32), inc[:-k]], axis=0)
        k *= 2
    out_ref[0] = carry_ref[...] + (inc - x)
    carry_ref[...] = carry_ref[...] + inc[CS_CH - 1:CS_CH, :]


def _prefix_table(x, nb):
    return pl.pallas_call(
        _cumsum_body,
        grid=(nb, L // CS_CH),
        in_specs=[pl.BlockSpec((1, CS_CH, H), lambda b, c: (b, c, 0))],
        out_specs=pl.BlockSpec((1, CS_CH, H), lambda b, c: (b, c, 0)),
        out_shape=jax.ShapeDtypeStruct((nb, L, H), jnp.float32),
        scratch_shapes=[pltpu.VMEM((1, H), jnp.float32)],
    )(x)


def _make_sc_pool_body(total, wpt, nchunk):
    def body(table, s_hbm, e_hbm, pooled_hbm, inv_hbm, mask_hbm,
             s_v, e_v, idx_s, idx_e, inv_v, mask_v,
             rows_s, rows_e, pooled_v, sem_s, sem_e):
        wid = lax.axis_index("s") * NC + lax.axis_index("c")
        base = wid * wpt
        brow = (base // W) * L

        pltpu.sync_copy(s_hbm.at[pl.ds(base, wpt)], s_v)
        pltpu.sync_copy(e_hbm.at[pl.ds(base, wpt)], e_v)

        for g in range(wpt // LANES):
            sl = pl.ds(g * LANES, LANES)
            vs = s_v[sl]
            ve = e_v[sl]
            valid = vs < ve
            cnt = jnp.where(valid, ve - vs, 1).astype(jnp.float32)
            inv_v[sl] = 1.0 / cnt
            mask_v[sl] = jnp.where(valid, 1.0, 0.0)
            c0, r0 = divmod(g * LANES, CW)
            idx_s[c0, pl.ds(r0, LANES)] = vs + brow
            idx_e[c0, pl.ds(r0, LANES)] = ve + brow

        pltpu.sync_copy(inv_v, inv_hbm.at[pl.ds(base, wpt)])
        pltpu.sync_copy(mask_v, mask_hbm.at[pl.ds(base, wpt)])

        for c in range(nchunk):
            cp_s = pltpu.async_copy(table.at[idx_s.at[c]], rows_s, sem_s)
            cp_e = pltpu.async_copy(table.at[idx_e.at[c]], rows_e, sem_e)
            cp_s.wait()
            cp_e.wait()

            def word(i, carry):
                def hchunk(k, carry2):
                    hs = pl.ds(k * LANES, LANES)
                    pooled_v[i, hs] = rows_e[i, hs] - rows_s[i, hs]
                    return carry2
                return lax.fori_loop(0, H // LANES, hchunk, carry)

            lax.fori_loop(0, CW, word, 0)
            pltpu.sync_copy(pooled_v, pooled_hbm.at[pl.ds(base + c * CW, CW)])
    return body


def _sc_pool(table, s_flat, e_flat):
    total = s_flat.shape[0]
    wpt = total // NW
    nchunk = wpt // CW
    mesh = plsc.VectorSubcoreMesh(core_axis_name="c", subcore_axis_name="s",
                                  num_cores=NC, num_subcores=NS)
    fn = functools.partial(
        pl.kernel,
        out_type=(jax.ShapeDtypeStruct((total, H), jnp.float32),
                  jax.ShapeDtypeStruct((total,), jnp.float32),
                  jax.ShapeDtypeStruct((total,), jnp.float32)),
        mesh=mesh,
        scratch_types=[
            pltpu.VMEM((wpt,), jnp.int32),
            pltpu.VMEM((wpt,), jnp.int32),
            pltpu.VMEM((nchunk, CW), jnp.int32),
            pltpu.VMEM((nchunk, CW), jnp.int32),
            pltpu.VMEM((wpt,), jnp.float32),
            pltpu.VMEM((wpt,), jnp.float32),
            pltpu.VMEM((CW, H), jnp.float32),
            pltpu.VMEM((CW, H), jnp.float32),
            pltpu.VMEM((CW, H), jnp.float32),
            pltpu.SemaphoreType.DMA,
            pltpu.SemaphoreType.DMA,
        ],
    )(_make_sc_pool_body(total, wpt, nchunk))
    return fn(table, s_flat, e_flat)


def _mlp_body(p_ref, inv_ref, w1_ref, b1_ref, w2_ref, b2_ref, out_ref):
    p = p_ref[...] * inv_ref[...]
    h = jnp.dot(p, w1_ref[...], preferred_element_type=jnp.float32)
    h = h + b1_ref[...]
    h = 0.5 * h * (1.0 + lax.erf(h * (2.0 ** -0.5)))
    o = jnp.dot(h, w2_ref[...], preferred_element_type=jnp.float32)
    out_ref[...] = o + b2_ref[...]


def _mlp(pooled_sum, inv, w1, b1, w2, b2):
    total = pooled_sum.shape[0]
    return pl.pallas_call(
        _mlp_body,
        grid=(total // MLP_BLK,),
        in_specs=[
            pl.BlockSpec((MLP_BLK, H), lambda i: (i, 0)),
            pl.BlockSpec((MLP_BLK, 1), lambda i: (i, 0)),
            pl.BlockSpec((H, HID), lambda i: (0, 0)),
            pl.BlockSpec((1, HID), lambda i: (0, 0)),
            pl.BlockSpec((HID, C), lambda i: (0, 0)),
            pl.BlockSpec((1, C), lambda i: (0, 0)),
        ],
        out_specs=pl.BlockSpec((MLP_BLK, C), lambda i: (i, 0)),
        out_shape=jax.ShapeDtypeStruct((total, C), jnp.float32),
    )(pooled_sum, inv, w1, b1, w2, b2)


def kernel(encoder_output, word_offsets, W1, b1, W2, b2):
    b1r = b1.reshape(1, HID)
    b2r = b2.reshape(1, C)
    pooled_parts, inv_parts, mask_parts = [], [], []
    for i in range(NSPLIT):
        xs = encoder_output[i * BSL:(i + 1) * BSL]
        wo = word_offsets[i * BSL:(i + 1) * BSL]
        table = _prefix_table(xs, BSL).reshape(BSL * L, H)
        s_flat = wo[..., 0].reshape(BSL * W)
        e_flat = wo[..., 1].reshape(BSL * W)
        pooled_sum, inv, mask = _sc_pool(table, s_flat, e_flat)
        pooled_parts.append(pooled_sum)
        inv_parts.append(inv)
        mask_parts.append(mask)
    logits_parts = [
        _mlp(p, iv.reshape(-1, 1), W1, b1r, W2, b2r)
        for p, iv in zip(pooled_parts, inv_parts)
    ]
    logits = jnp.concatenate(logits_parts, axis=0)
    mask = jnp.concatenate(mask_parts, axis=0)
    return logits.reshape(B, W, C), mask.reshape(B, W)


# speedup vs baseline: 1.3700x; 1.3700x over previous
import functools

import jax
import jax.numpy as jnp
from jax import lax
from jax.experimental import pallas as pl
from jax.experimental.pallas import tpu as pltpu
from jax.experimental.pallas import tpu_sc as plsc

B, L, H, W, C = 16, 4096, 512, 512, 32
HID = H // 2

NC, NS, LANES = 2, 16, 16
NW = NC * NS
CW = 64

NSPLIT = 2
BSL = B // NSPLIT

CS_CH = 512
MLP_BLK = 512


def _cumsum_body(x_ref, out_ref, carry_ref):
    c = pl.program_id(1)

    @pl.when(c == 0)
    def _():
        carry_ref[...] = jnp.zeros_like(carry_ref)

    x = x_ref[0]
    inc = x
    k = 1
    while k < CS_CH:
        inc = inc + jnp.concatenate(
            [jnp.zeros((k, H), jnp.float32), inc[:-k]], axis=0)
        k *= 2
    out_ref[0] = carry_ref[...] + (inc - x)
    carry_ref[...] = carry_ref[...] + inc[CS_CH - 1:CS_CH, :]


def _prefix_table(x, nb, b0):
    return pl.pallas_call(
        _cumsum_body,
        grid=(nb, L // CS_CH),
        in_specs=[pl.BlockSpec((1, CS_CH, H), lambda b, c: (b + b0, c, 0))],
        out_specs=pl.BlockSpec((1, CS_CH, H), lambda b, c: (b, c, 0)),
        out_shape=jax.ShapeDtypeStruct((nb, L, H), jnp.float32),
        scratch_shapes=[pltpu.VMEM((1, H), jnp.float32)],
    )(x)


def _make_sc_pool_body(total, wpt, nchunk):
    def body(table, s_hbm, e_hbm, pooled_hbm, inv_hbm, mask_hbm,
             s_v, e_v, idx_s, idx_e, inv_v, mask_v,
             rows_s, rows_e, pooled_v, sem_s, sem_e):
        wid = lax.axis_index("s") * NC + lax.axis_index("c")
        base = wid * wpt
        brow = (base // W) * L

        pltpu.sync_copy(s_hbm.at[pl.ds(base, wpt)], s_v)
        pltpu.sync_copy(e_hbm.at[pl.ds(base, wpt)], e_v)

        for g in range(wpt // LANES):
            sl = pl.ds(g * LANES, LANES)
            vs = s_v[sl]
            ve = e_v[sl]
            valid = vs < ve
            cnt = jnp.where(valid, ve - vs, 1).astype(jnp.float32)
            inv_v[sl] = 1.0 / cnt
            mask_v[sl] = jnp.where(valid, 1.0, 0.0)
            c0, r0 = divmod(g * LANES, CW)
            idx_s[c0, pl.ds(r0, LANES)] = vs + brow
            idx_e[c0, pl.ds(r0, LANES)] = ve + brow

        pltpu.sync_copy(inv_v, inv_hbm.at[pl.ds(base, wpt)])
        pltpu.sync_copy(mask_v, mask_hbm.at[pl.ds(base, wpt)])

        for c in range(nchunk):
            cp_s = pltpu.async_copy(table.at[idx_s.at[c]], rows_s, sem_s)
            cp_e = pltpu.async_copy(table.at[idx_e.at[c]], rows_e, sem_e)
            cp_s.wait()
            cp_e.wait()

            def word(i, carry):
                def hchunk(k, carry2):
                    hs = pl.ds(k * LANES, LANES)
                    pooled_v[i, hs] = rows_e[i, hs] - rows_s[i, hs]
                    return carry2
                return lax.fori_loop(0, H // LANES, hchunk, carry)

            lax.fori_loop(0, CW, word, 0)
            pltpu.sync_copy(pooled_v, pooled_hbm.at[pl.ds(base + c * CW, CW)])
    return body


def _sc_pool(table, s_flat, e_flat):
    total = s_flat.shape[0]
    wpt = total // NW
    nchunk = wpt // CW
    mesh = plsc.VectorSubcoreMesh(core_axis_name="c", subcore_axis_name="s",
                                  num_cores=NC, num_subcores=NS)
    fn = functools.partial(
        pl.kernel,
        out_type=(jax.ShapeDtypeStruct((total, H), jnp.float32),
                  jax.ShapeDtypeStruct((total,), jnp.float32),
                  jax.ShapeDtypeStruct((total,), jnp.float32)),
        mesh=mesh,
        scratch_types=[
            pltpu.VMEM((wpt,), jnp.int32),
            pltpu.VMEM((wpt,), jnp.int32),
            pltpu.VMEM((nchunk, CW), jnp.int32),
            pltpu.VMEM((nchunk, CW), jnp.int32),
            pltpu.VMEM((wpt,), jnp.float32),
            pltpu.VMEM((wpt,), jnp.float32),
            pltpu.VMEM((CW, H), jnp.float32),
            pltpu.VMEM((CW, H), jnp.float32),
            pltpu.VMEM((CW, H), jnp.float32),
            pltpu.SemaphoreType.DMA,
            pltpu.SemaphoreType.DMA,
        ],
    )(_make_sc_pool_body(total, wpt, nchunk))
    return fn(table, s_flat, e_flat)


def _mlp_body(p_ref, inv_ref, w1_ref, b1_ref, w2_ref, b2_ref, out_ref):
    p = p_ref[...] * inv_ref[...]
    h = jnp.dot(p, w1_ref[...], preferred_element_type=jnp.float32)
    h = h + b1_ref[...]
    h = 0.5 * h * (1.0 + lax.erf(h * (2.0 ** -0.5)))
    o = jnp.dot(h, w2_ref[...], preferred_element_type=jnp.float32)
    out_ref[...] = o + b2_ref[...]


def _mlp(pooled_sum, inv, w1, b1, w2, b2):
    total = pooled_sum.shape[0]
    return pl.pallas_call(
        _mlp_body,
        grid=(total // MLP_BLK,),
        in_specs=[
            pl.BlockSpec((MLP_BLK, H), lambda i: (i, 0)),
            pl.BlockSpec((MLP_BLK, 1), lambda i: (i, 0)),
            pl.BlockSpec((H, HID), lambda i: (0, 0)),
            pl.BlockSpec((1, HID), lambda i: (0, 0)),
            pl.BlockSpec((HID, C), lambda i: (0, 0)),
            pl.BlockSpec((1, C), lambda i: (0, 0)),
        ],
        out_specs=pl.BlockSpec((MLP_BLK, C), lambda i: (i, 0)),
        out_shape=jax.ShapeDtypeStruct((total, C), jnp.float32),
    )(pooled_sum, inv, w1, b1, w2, b2)


def kernel(encoder_output, word_offsets, W1, b1, W2, b2):
    b1r = b1.reshape(1, HID)
    b2r = b2.reshape(1, C)
    pooled_parts, inv_parts, mask_parts = [], [], []
    for i in range(NSPLIT):
        wo = word_offsets[i * BSL:(i + 1) * BSL]
        table = _prefix_table(encoder_output, BSL, i * BSL).reshape(BSL * L, H)
        s_flat = wo[..., 0].reshape(BSL * W)
        e_flat = wo[..., 1].reshape(BSL * W)
        pooled_sum, inv, mask = _sc_pool(table, s_flat, e_flat)
        pooled_parts.append(pooled_sum)
        inv_parts.append(inv)
        mask_parts.append(mask)
    logits_parts = [
        _mlp(p, iv.reshape(-1, 1), W1, b1r, W2, b2r)
        for p, iv in zip(pooled_parts, inv_parts)
    ]
    logits = jnp.concatenate(logits_parts, axis=0)
    mask = jnp.concatenate(mask_parts, axis=0)
    return logits.reshape(B, W, C), mask.reshape(B, W)
